# baseline (speedup 1.0000x reference)
import functools

import jax
import jax.numpy as jnp
from jax import lax
from jax.experimental import pallas as pl
from jax.experimental.pallas import tpu as pltpu
from jax.experimental.pallas import tpu_sc as plsc

NC = 2
NS = 16
NW = NC * NS
CHE = 2000
K = 10
SUPE = K * 128
BE2 = 16000

_F32 = jnp.float32
_I32 = jnp.int32


def _iota16():
    return jnp.arange(16, dtype=_I32)


def _splat(ref, i):
    return plsc.load_gather(ref, [jnp.full((16,), i, _I32)])


def _p1_body(n_edges, xT, src_i, trg_i, cvec, srcT, dparts,
             row0, row1, cv,
             idxs0, idxt0, idxs1, idxt1,
             sb0_0, sb1_0, db_0, sb0_1, sb1_1, db_1,
             sem_i0, sem_i1, sem_w0, sem_w1):
    nchunk = n_edges // CHE
    cid = lax.axis_index("c")
    sid = lax.axis_index("s")
    wid = sid * NC + cid
    h = wid // 8
    fg = wid % 8
    r0 = h * 16 + 2 * fg
    r1 = r0 + 1

    f0 = 2 * fg
    f1 = f0 + 1
    pltpu.sync_copy(xT.at[r0], row0)
    pltpu.sync_copy(xT.at[r1], row1)
    pltpu.sync_copy(cvec, cv)
    c0 = _splat(cv, r0)
    c1 = _splat(cv, r1)

    idx_sets = ((idxs0, idxt0, sem_i0), (idxs1, idxt1, sem_i1))
    wb_sets = ((sb0_0, sb1_0, db_0, sem_w0), (sb0_1, sb1_1, db_1, sem_w1))

    pltpu.async_copy(src_i.at[pl.ds(0, CHE)], idxs0, sem_i0)
    pltpu.async_copy(trg_i.at[pl.ds(0, CHE)], idxt0, sem_i0)

    def outer(t, carry):
        for par in (0, 1):
            j = 2 * t + par
            b = pl.multiple_of(j * CHE, CHE)
            isb, itb, semi = idx_sets[par]
            nsb, ntb, semn = idx_sets[1 - par]
            s0b, s1b, dbf, semw = wb_sets[par]

            @pl.when(j + 1 < nchunk)
            def _():
                bn = pl.multiple_of((j + 1) * CHE, CHE)
                pltpu.async_copy(src_i.at[pl.ds(bn, CHE)], nsb, semn)
                pltpu.async_copy(trg_i.at[pl.ds(bn, CHE)], ntb, semn)

            @pl.when(j >= 2)
            def _():
                pltpu.make_async_copy(s0b, srcT.at[h, f0, pl.ds(0, CHE)], semw).wait()
                pltpu.make_async_copy(s1b, srcT.at[h, f1, pl.ds(0, CHE)], semw).wait()
                pltpu.make_async_copy(dbf, dparts.at[fg, h, pl.ds(0, CHE)], semw).wait()

            pltpu.make_async_copy(src_i.at[pl.ds(0, CHE)], isb, semi).wait()
            pltpu.make_async_copy(trg_i.at[pl.ds(0, CHE)], itb, semi).wait()

            def inner(i, c2):
                sl = pl.ds(i * 16, 16)
                ivs = isb[sl]
                ivt = itb[sl]
                s0 = plsc.load_gather(row0, [ivs])
                t0 = plsc.load_gather(row0, [ivt])
                s1 = plsc.load_gather(row1, [ivs])
                t1 = plsc.load_gather(row1, [ivt])
                d0 = t0 - s0
                d1 = t1 - s1
                s0b[sl] = s0
                s1b[sl] = s1
                dbf[sl] = c0 * d0 * d0 + c1 * d1 * d1
                return c2

            lax.fori_loop(0, CHE // 16, inner, 0)

            pltpu.async_copy(s0b, srcT.at[h, f0, pl.ds(b, CHE)], semw)
            pltpu.async_copy(s1b, srcT.at[h, f1, pl.ds(b, CHE)], semw)
            pltpu.async_copy(dbf, dparts.at[fg, h, pl.ds(b, CHE)], semw)
        return carry

    lax.fori_loop(0, nchunk // 2, outer, 0)
    for s0b, s1b, dbf, semw in wb_sets:
        pltpu.make_async_copy(s0b, srcT.at[h, f0, pl.ds(0, CHE)], semw).wait()
        pltpu.make_async_copy(s1b, srcT.at[h, f1, pl.ds(0, CHE)], semw).wait()
        pltpu.make_async_copy(dbf, dparts.at[fg, h, pl.ds(0, CHE)], semw).wait()


def _p2_body(dp_ref, dT_ref, sum_ref, min_ref):
    i = pl.program_id(0)
    acc = dp_ref[0, ...]
    for k in range(1, 8):
        acc = acc + dp_ref[k, ...]
    dT_ref[...] = acc
    ps = jnp.sum(acc, axis=1, keepdims=True)
    pm = jnp.min(acc, axis=1, keepdims=True)

    @pl.when(i == 0)
    def _():
        sum_ref[...] = ps
        min_ref[...] = pm

    @pl.when(i != 0)
    def _():
        sum_ref[...] += ps
        min_ref[...] = jnp.minimum(min_ref[...], pm)


def _p2b_body(d_ref, mean_ref, shift_ref, e_ref):
    x = d_ref[...] + mean_ref[...]
    lk = jnp.where(x >= 0.0, x, 0.2 * x)
    e_ref[...] = jnp.exp(-lk - shift_ref[...])


def _p3_body(n_nodes, n_edges, eT, trg2d, zrows, parts,
             idxb, d0b, d1b, d2b, d3b, vals, shared, sem_s):
    nsup = n_edges // SUPE
    niter = (nsup + NW - 1) // NW
    cid = lax.axis_index("c")
    sid = lax.axis_index("s")
    wid = sid * NC + cid
    rpt = n_nodes // NS
    dbufs = (d0b, d1b, d2b, d3b)

    pltpu.sync_copy(zrows, shared.at[pl.ds(sid * rpt, rpt)])
    pltpu.sync_copy(zrows.at[pl.ds(0, SUPE)], vals)
    plsc.subcore_barrier()

    def body(j, carry):
        sup = wid + j * NW

        @pl.when(sup < nsup)
        def _():
            b = pl.multiple_of(sup * SUPE, SUPE)
            pltpu.sync_copy(trg2d.at[pl.ds(sup * K, K)], idxb)
            for hh in range(4):
                pltpu.sync_copy(eT.at[hh, pl.ds(b, SUPE)], dbufs[hh])
            for hh in range(4):
                def inner(i, c2, hh=hh):
                    ev = dbufs[hh][pl.ds(i * 16, 16)]
                    plsc.store_scatter(
                        vals, [_iota16() + i * 16, jnp.full((16,), hh, _I32)], ev)
                    return c2
                lax.fori_loop(0, SUPE // 16, inner, 0)
            for k in range(K):
                pltpu.async_copy(vals.at[pl.ds(k * 128, 128)],
                                 shared.at[idxb.at[k]], sem_s, add=True)
            for k in range(K):
                pltpu.make_async_copy(vals.at[pl.ds(0, 128)],
                                      shared.at[pl.ds(0, 128)], sem_s).wait()

        return carry

    lax.fori_loop(0, niter, body, 0)
    plsc.subcore_barrier()
    pltpu.sync_copy(shared.at[pl.ds(sid * rpt, rpt)],
                    parts.at[cid, pl.ds(sid * rpt, rpt)])


def _p4_body(a_ref, b_ref, o_ref):
    o_ref[...] = a_ref[...] + b_ref[...]


def _p5_body(n_edges, eT, trg2d, nbr, attT,
             idxb, d0b, d1b, d2b, d3b, denb, a0b, a1b, a2b, a3b, sem_g):
    nsup = n_edges // SUPE
    niter = (nsup + NW - 1) // NW
    cid = lax.axis_index("c")
    sid = lax.axis_index("s")
    wid = sid * NC + cid
    dbufs = (d0b, d1b, d2b, d3b)
    abufs = (a0b, a1b, a2b, a3b)

    def body(j, carry):
        sup = wid + j * NW

        @pl.when(sup < nsup)
        def _():
            b = pl.multiple_of(sup * SUPE, SUPE)
            pltpu.sync_copy(trg2d.at[pl.ds(sup * K, K)], idxb)
            for k in range(K):
                pltpu.async_copy(nbr.at[idxb.at[k]],
                                 denb.at[pl.ds(k * 128, 128)], sem_g)
            for hh in range(4):
                pltpu.sync_copy(eT.at[hh, pl.ds(b, SUPE)], dbufs[hh])
            for k in range(K):
                pltpu.make_async_copy(nbr.at[pl.ds(0, 128)],
                                      denb.at[pl.ds(0, 128)], sem_g).wait()
            for hh in range(4):
                def inner(i, c2, hh=hh):
                    ev = dbufs[hh][pl.ds(i * 16, 16)]
                    den = plsc.load_gather(
                        denb, [_iota16() + i * 16, jnp.full((16,), hh, _I32)])
                    abufs[hh][pl.ds(i * 16, 16)] = ev / (den + 1e-16)
                    return c2
                lax.fori_loop(0, SUPE // 16, inner, 0)
            for hh in range(4):
                pltpu.sync_copy(abufs[hh], attT.at[hh, pl.ds(b, SUPE)])

        return carry

    lax.fori_loop(0, niter, body, 0)


def kernel(nodes_features, edge_index, edge_dims_weights, distance_dims_weights):
    n_nodes, n_heads, n_feat = nodes_features.shape
    n_edges = edge_index.shape[1]
    hf = n_heads * n_feat
    assert hf == 64 and n_heads == 4 and n_feat == 16
    assert n_edges % (2 * CHE) == 0 and n_edges % SUPE == 0
    assert n_nodes % NS == 0

    trg = edge_index[0]
    src = edge_index[1]
    trg2d = trg.reshape(n_edges // 128, 128)
    xT = jnp.transpose(nodes_features, (1, 2, 0)).reshape(hf, n_nodes)
    cvec = (distance_dims_weights * edge_dims_weights * edge_dims_weights
            ).reshape(hf)

    mesh = plsc.VectorSubcoreMesh(core_axis_name="c", subcore_axis_name="s",
                                  num_cores=NC, num_subcores=NS)
    sc_params = pltpu.CompilerParams(use_tc_tiling_on_sc=False,
                                     needs_layout_passes=False)

    p1 = pl.kernel(
        functools.partial(_p1_body, n_edges),
        out_type=[jax.ShapeDtypeStruct((n_heads, n_feat, n_edges), _F32),
                  jax.ShapeDtypeStruct((8, 4, n_edges), _F32)],
        mesh=mesh,
        scratch_types=[pltpu.VMEM((n_nodes,), _F32),
                       pltpu.VMEM((n_nodes,), _F32),
                       pltpu.VMEM((hf,), _F32),
                       pltpu.VMEM((CHE,), _I32), pltpu.VMEM((CHE,), _I32),
                       pltpu.VMEM((CHE,), _I32), pltpu.VMEM((CHE,), _I32),
                       pltpu.VMEM((CHE,), _F32), pltpu.VMEM((CHE,), _F32),
                       pltpu.VMEM((CHE,), _F32), pltpu.VMEM((CHE,), _F32),
                       pltpu.VMEM((CHE,), _F32), pltpu.VMEM((CHE,), _F32),
                       pltpu.SemaphoreType.DMA, pltpu.SemaphoreType.DMA,
                       pltpu.SemaphoreType.DMA, pltpu.SemaphoreType.DMA],
        compiler_params=sc_params,
    )
    srcT, dparts = p1(xT, src, trg, cvec)

    dT, sums, mins = pl.pallas_call(
        _p2_body,
        grid=(n_edges // BE2,),
        in_specs=[pl.BlockSpec((8, 4, BE2), lambda i: (0, 0, i))],
        out_specs=[pl.BlockSpec((4, BE2), lambda i: (0, i)),
                   pl.BlockSpec((4, 1), lambda i: (0, 0)),
                   pl.BlockSpec((4, 1), lambda i: (0, 0))],
        out_shape=[jax.ShapeDtypeStruct((4, n_edges), _F32),
                   jax.ShapeDtypeStruct((4, 1), _F32),
                   jax.ShapeDtypeStruct((4, 1), _F32)],
    )(dparts)

    mean4 = sums[:, 0] / n_edges
    t4 = mins[:, 0] + mean4
    lk4 = jnp.where(t4 >= 0.0, t4, 0.2 * t4)
    shift = jnp.max(-lk4)
    eT = pl.pallas_call(
        _p2b_body,
        grid=(n_edges // BE2,),
        in_specs=[pl.BlockSpec((4, BE2), lambda i: (0, i)),
                  pl.BlockSpec((4, 1), lambda i: (0, 0)),
                  pl.BlockSpec((4, 1), lambda i: (0, 0))],
        out_specs=pl.BlockSpec((4, BE2), lambda i: (0, i)),
        out_shape=jax.ShapeDtypeStruct((4, n_edges), _F32),
    )(dT, mean4[:, None], jnp.full((4, 1), shift, _F32))

    rpt = n_nodes // NS
    zrows = jnp.zeros((rpt, 8), dtype=_F32)
    p3 = pl.kernel(
        functools.partial(_p3_body, n_nodes, n_edges),
        out_type=jax.ShapeDtypeStruct((NC, n_nodes, 8), _F32),
        mesh=mesh,
        scratch_types=[pltpu.VMEM((K, 128), _I32),
                       pltpu.VMEM((SUPE,), _F32), pltpu.VMEM((SUPE,), _F32),
                       pltpu.VMEM((SUPE,), _F32), pltpu.VMEM((SUPE,), _F32),
                       pltpu.VMEM((SUPE, 8), _F32),
                       pltpu.VMEM_SHARED((n_nodes, 8), _F32),
                       pltpu.SemaphoreType.DMA],
        compiler_params=sc_params,
    )
    parts = p3(eT, trg2d, zrows)

    nr = n_nodes * 8 // 128
    nbr = pl.pallas_call(
        _p4_body,
        out_shape=jax.ShapeDtypeStruct((nr, 128), _F32),
    )(parts[0].reshape(nr, 128), parts[1].reshape(nr, 128)
      ).reshape(n_nodes, 8)

    p5 = pl.kernel(
        functools.partial(_p5_body, n_edges),
        out_type=jax.ShapeDtypeStruct((4, n_edges), _F32),
        mesh=mesh,
        scratch_types=[pltpu.VMEM((K, 128), _I32),
                       pltpu.VMEM((SUPE,), _F32), pltpu.VMEM((SUPE,), _F32),
                       pltpu.VMEM((SUPE,), _F32), pltpu.VMEM((SUPE,), _F32),
                       pltpu.VMEM((SUPE, 8), _F32),
                       pltpu.VMEM((SUPE,), _F32), pltpu.VMEM((SUPE,), _F32),
                       pltpu.VMEM((SUPE,), _F32), pltpu.VMEM((SUPE,), _F32),
                       pltpu.SemaphoreType.DMA],
        compiler_params=sc_params,
    )
    attT = p5(eT, trg2d, nbr)

    attentions = jnp.transpose(attT, (1, 0))[:, :, None]
    nfs = jnp.transpose(srcT, (2, 0, 1))
    return attentions, nfs


# speedup vs baseline: 2.6405x; 2.6405x over previous
import functools

import jax
import jax.numpy as jnp
import numpy as np
from jax import lax
from jax.experimental import pallas as pl
from jax.experimental.pallas import tpu as pltpu
from jax.experimental.pallas import tpu_sc as plsc

NC = 2
NS = 16
NW = NC * NS
CH = 128

_F32 = jnp.float32


def _sel_matrix() -> np.ndarray:
    s = np.zeros((128, 16), dtype=np.float32)
    for l in range(128):
        s[l, 8 * (l // 64) + (l % 64) // 16] = 1.0
    return s


_SEL = _sel_matrix()


def _sc_gather2_body(nchunks, x64, src_i, trg_i, src_out, trg_out,
                     idx_s, idx_t, rows_s, rows_t, sem_s, sem_t):
    wid = lax.axis_index("s") * NC + lax.axis_index("c")
    niter = (nchunks + NW - 1) // NW

    def body(j, carry):
        cidx = wid + j * NW

        @pl.when(cidx < nchunks)
        def _():
            b = pl.multiple_of(cidx * CH, CH)
            pltpu.sync_copy(src_i.at[pl.ds(b, CH)], idx_s)
            pltpu.sync_copy(trg_i.at[pl.ds(b, CH)], idx_t)
            cs = pltpu.async_copy(x64.at[idx_s], rows_s, sem_s)
            ct = pltpu.async_copy(x64.at[idx_t], rows_t, sem_t)
            cs.wait()
            ct.wait()
            pltpu.sync_copy(rows_s, src_out.at[pl.ds(b, CH)])
            pltpu.sync_copy(rows_t, trg_out.at[pl.ds(b, CH)])

        return carry

    lax.fori_loop(0, niter, body, 0)


def _sc_scatter_body(nchunks, n_nodes, exp8, trg_i, zrows, parts,
                     idx, vals, shared):
    cid = lax.axis_index("c")
    sid = lax.axis_index("s")
    wid = sid * NC + cid
    rpt = n_nodes // NS
    niter = (nchunks + NW - 1) // NW

    pltpu.sync_copy(zrows, shared.at[pl.ds(sid * rpt, rpt)])
    plsc.subcore_barrier()

    def body(j, carry):
        cidx = wid + j * NW

        @pl.when(cidx < nchunks)
        def _():
            b = pl.multiple_of(cidx * CH, CH)
            pltpu.sync_copy(trg_i.at[pl.ds(b, CH)], idx)
            pltpu.sync_copy(exp8.at[pl.ds(b, CH)], vals)
            pltpu.sync_copy(vals, shared.at[idx], add=True)

        return carry

    lax.fori_loop(0, niter, body, 0)
    plsc.subcore_barrier()
    pltpu.sync_copy(shared.at[pl.ds(sid * rpt, rpt)],
                    parts.at[cid, pl.ds(sid * rpt, rpt)])


def _sc_gather1_body(nchunks, nbr8, trg_i, out8, idx, rows, sem):
    wid = lax.axis_index("s") * NC + lax.axis_index("c")
    niter = (nchunks + NW - 1) // NW

    def body(j, carry):
        cidx = wid + j * NW

        @pl.when(cidx < nchunks)
        def _():
            b = pl.multiple_of(cidx * CH, CH)
            pltpu.sync_copy(trg_i.at[pl.ds(b, CH)], idx)
            pltpu.async_copy(nbr8.at[idx], rows, sem).wait()
            pltpu.sync_copy(rows, out8.at[pl.ds(b, CH)])

        return carry

    lax.fori_loop(0, niter, body, 0)


def _tc_dist_body(s_ref, t_ref, c_ref, sel_ref, d_ref, sum_ref, min_ref):
    i = pl.program_id(0)
    diff = t_ref[...] - s_ref[...]
    w2 = diff * diff * c_ref[...]
    d16 = jnp.dot(w2, sel_ref[...], preferred_element_type=_F32)
    d_ref[...] = d16
    psum = jnp.sum(d16, axis=0, keepdims=True)
    pmin = jnp.min(d16, axis=0, keepdims=True)

    @pl.when(i == 0)
    def _():
        sum_ref[...] = psum
        min_ref[...] = pmin

    @pl.when(i != 0)
    def _():
        sum_ref[...] += psum
        min_ref[...] = jnp.minimum(min_ref[...], pmin)


def _tc_exp_body(d_ref, mean_ref, m_ref, e_ref):
    x = d_ref[...] + mean_ref[...]
    lk = jnp.where(x >= 0.0, x, 0.2 * x)
    e_ref[...] = jnp.exp(-lk - m_ref[...])


def _tc_add_body(a_ref, b_ref, o_ref):
    o_ref[...] = a_ref[...] + b_ref[...]


def _tc_div_body(e_ref, dn_ref, o_ref):
    o_ref[...] = e_ref[...] / (dn_ref[...] + 1e-16)


def kernel(nodes_features, edge_index, edge_dims_weights, distance_dims_weights):
    n_nodes, n_heads, n_feat = nodes_features.shape
    n_edges = edge_index.shape[1]
    hf = n_heads * n_feat
    assert hf == 64 and n_heads == 4 and n_feat == 16
    assert n_edges % (2 * CH) == 0 and n_nodes % NS == 0

    nchunks = n_edges // CH
    trg = edge_index[0]
    src = edge_index[1]
    x64 = nodes_features.reshape(n_nodes, hf)

    mesh = plsc.VectorSubcoreMesh(core_axis_name="c", subcore_axis_name="s",
                                  num_cores=NC, num_subcores=NS)
    sc_params = pltpu.CompilerParams(use_tc_tiling_on_sc=False)

    gather2 = pl.kernel(
        functools.partial(_sc_gather2_body, nchunks),
        out_type=[jax.ShapeDtypeStruct((n_edges, hf), _F32),
                  jax.ShapeDtypeStruct((n_edges, hf), _F32)],
        mesh=mesh,
        scratch_types=[pltpu.VMEM((CH,), jnp.int32),
                       pltpu.VMEM((CH,), jnp.int32),
                       pltpu.VMEM((CH, hf), _F32),
                       pltpu.VMEM((CH, hf), _F32),
                       pltpu.SemaphoreType.DMA,
                       pltpu.SemaphoreType.DMA],
        compiler_params=sc_params,
    )
    src_rows, trg_rows = gather2(x64, src, trg)

    e2 = n_edges // 2
    be2 = 4000
    assert e2 % be2 == 0
    cw = (distance_dims_weights * edge_dims_weights * edge_dims_weights)
    c128 = jnp.tile(cw.reshape(1, hf), (1, 2))
    d16, sums16, mins16 = pl.pallas_call(
        _tc_dist_body,
        grid=(e2 // be2,),
        in_specs=[pl.BlockSpec((be2, 128), lambda i: (i, 0)),
                  pl.BlockSpec((be2, 128), lambda i: (i, 0)),
                  pl.BlockSpec((1, 128), lambda i: (0, 0)),
                  pl.BlockSpec((128, 16), lambda i: (0, 0))],
        out_specs=[pl.BlockSpec((be2, 16), lambda i: (i, 0)),
                   pl.BlockSpec((1, 16), lambda i: (0, 0)),
                   pl.BlockSpec((1, 16), lambda i: (0, 0))],
        out_shape=[jax.ShapeDtypeStruct((e2, 16), _F32),
                   jax.ShapeDtypeStruct((1, 16), _F32),
                   jax.ShapeDtypeStruct((1, 16), _F32)],
    )(src_rows.reshape(e2, 128), trg_rows.reshape(e2, 128), c128,
      jnp.asarray(_SEL))

    s16 = sums16[0]
    m16 = mins16[0]
    mean8 = (s16[:8] + s16[8:]) / n_edges
    t8 = jnp.minimum(m16[:8], m16[8:]) + mean8
    lk8 = jnp.where(t8 >= 0.0, t8, 0.2 * t8)
    mshift = jnp.max(-lk8)
    mean128 = jnp.tile(mean8, 16).reshape(1, 128)
    m128 = jnp.full((1, 128), mshift, dtype=_F32)

    e16 = n_edges // 16
    be3 = 5000
    assert e16 % be3 == 0
    e128 = pl.pallas_call(
        _tc_exp_body,
        grid=(e16 // be3,),
        in_specs=[pl.BlockSpec((be3, 128), lambda i: (i, 0)),
                  pl.BlockSpec((1, 128), lambda i: (0, 0)),
                  pl.BlockSpec((1, 128), lambda i: (0, 0))],
        out_specs=pl.BlockSpec((be3, 128), lambda i: (i, 0)),
        out_shape=jax.ShapeDtypeStruct((e16, 128), _F32),
    )(d16.reshape(e16, 128), mean128, m128)
    exp8 = e128.reshape(n_edges, 8)

    rpt = n_nodes // NS
    zrows = jnp.zeros((rpt, 8), dtype=_F32)
    scatter = pl.kernel(
        functools.partial(_sc_scatter_body, nchunks, n_nodes),
        out_type=jax.ShapeDtypeStruct((NC, n_nodes, 8), _F32),
        mesh=mesh,
        scratch_types=[pltpu.VMEM((CH,), jnp.int32),
                       pltpu.VMEM((CH, 8), _F32),
                       pltpu.VMEM_SHARED((n_nodes, 8), _F32)],
        compiler_params=sc_params,
    )
    parts = scatter(exp8, trg, zrows)

    nr = n_nodes * 8 // 128
    nbr128 = pl.pallas_call(
        _tc_add_body,
        out_shape=jax.ShapeDtypeStruct((nr, 128), _F32),
    )(parts[0].reshape(nr, 128), parts[1].reshape(nr, 128))

    gatherd = pl.kernel(
        functools.partial(_sc_gather1_body, nchunks),
        out_type=jax.ShapeDtypeStruct((n_edges, 8), _F32),
        mesh=mesh,
        scratch_types=[pltpu.VMEM((CH,), jnp.int32),
                       pltpu.VMEM((CH, 8), _F32),
                       pltpu.SemaphoreType.DMA],
        compiler_params=sc_params,
    )
    denom8 = gatherd(nbr128.reshape(n_nodes, 8), trg)

    att128 = pl.pallas_call(
        _tc_div_body,
        grid=(e16 // be3,),
        in_specs=[pl.BlockSpec((be3, 128), lambda i: (i, 0)),
                  pl.BlockSpec((be3, 128), lambda i: (i, 0))],
        out_specs=pl.BlockSpec((be3, 128), lambda i: (i, 0)),
        out_shape=jax.ShapeDtypeStruct((e16, 128), _F32),
    )(e128, denom8.reshape(e16, 128))

    attentions = att128.reshape(n_edges, 8)[:, :4].reshape(n_edges, n_heads, 1)
    return attentions, src_rows.reshape(n_edges, n_heads, n_feat)
